# Initial kernel scaffold; baseline (speedup 1.0000x reference)
import jax
import jax.numpy as jnp
from jax.experimental import pallas as pl


def kernel(x, edge_index, edge_label_index, W_hl, b_hl, W_msg0, W_upd0, W_msg1, W_upd1):
    raise NotImplementedError("write your pallas kernel here")


# speedup vs baseline: 4.6374x; 4.6374x over previous
import jax
import jax.numpy as jnp
from jax import lax
from jax.experimental import pallas as pl
from jax.experimental.pallas import tpu as pltpu
from jax.experimental.pallas import tpu_sc as plsc

NC = 2
NS = 16
NW = NC * NS
CHUNK = 128
ZR = 8


def _zero_fill(ref, nrows, d):
    zero16 = jnp.zeros((16,), jnp.float32)

    @pl.loop(0, nrows)
    def _(i):
        @pl.loop(0, d // 16)
        def _(c):
            ref[i, pl.ds(c * 16, 16)] = zero16


def _sc_segment_sum(hm, edge_index):
    n, d = hm.shape
    e = edge_index.shape[1]
    nchunks = e // CHUNK
    iters = (nchunks + NW - 1) // NW
    nblk = n // ZR
    blk_iters = (nblk + NS - 1) // NS
    mesh = plsc.VectorSubcoreMesh(core_axis_name="c", subcore_axis_name="s")

    def body(hm_hbm, ei_hbm, out_hbm, srcv, dstv, rows, zbuf, acc, sem):
        cid = lax.axis_index("c")
        sid = lax.axis_index("s")
        wid = sid * NC + cid

        _zero_fill(zbuf, ZR, d)

        @pl.loop(0, blk_iters)
        def _(t):
            b = t * NS + sid

            @pl.when(b < nblk)
            def _():
                pltpu.sync_copy(zbuf, acc.at[pl.ds(b * ZR, ZR)])

        plsc.subcore_barrier()

        @pl.loop(0, iters)
        def _(j):
            g = j * NW + wid

            @pl.when(g < nchunks)
            def _():
                base = g * CHUNK
                pltpu.sync_copy(ei_hbm.at[0, pl.ds(base, CHUNK)], srcv)
                pltpu.sync_copy(ei_hbm.at[1, pl.ds(base, CHUNK)], dstv)
                pltpu.async_copy(hm_hbm.at[srcv], rows, sem).wait()
                pltpu.sync_copy(rows, acc.at[dstv], add=True)

        plsc.subcore_barrier()

        @pl.loop(0, blk_iters)
        def _(t):
            b = t * NS + sid

            @pl.when(b < nblk)
            def _():
                r0 = b * ZR
                pltpu.sync_copy(acc.at[pl.ds(r0, ZR)],
                                out_hbm.at[cid, pl.ds(r0, ZR)])

    k = pl.kernel(
        body,
        out_type=jax.ShapeDtypeStruct((NC, n, d), jnp.float32),
        mesh=mesh,
        scratch_types=[
            pltpu.VMEM((CHUNK,), jnp.int32),
            pltpu.VMEM((CHUNK,), jnp.int32),
            pltpu.VMEM((CHUNK, d), jnp.float32),
            pltpu.VMEM((ZR, d), jnp.float32),
            pltpu.VMEM_SHARED((n, d), jnp.float32),
            pltpu.SemaphoreType.DMA,
        ],
    )
    return k(hm, edge_index)


def _sc_degree(edge_index, n, d):
    e = edge_index.shape[1]
    nchunks = e // CHUNK
    iters = (nchunks + NW - 1) // NW
    nblk = n // ZR
    blk_iters = (nblk + NS - 1) // NS
    mesh = plsc.VectorSubcoreMesh(core_axis_name="c", subcore_axis_name="s")

    def body(ei_hbm, out_hbm, dstv, onesv, zbuf, acc, sem):
        cid = lax.axis_index("c")
        sid = lax.axis_index("s")
        wid = sid * NC + cid

        _zero_fill(zbuf, ZR, d)
        one16 = jnp.ones((16,), jnp.float32)

        @pl.loop(0, CHUNK)
        def _(i):
            @pl.loop(0, d // 16)
            def _(c):
                onesv[i, pl.ds(c * 16, 16)] = one16

        @pl.loop(0, blk_iters)
        def _(t):
            b = t * NS + sid

            @pl.when(b < nblk)
            def _():
                pltpu.sync_copy(zbuf, acc.at[pl.ds(b * ZR, ZR)])

        plsc.subcore_barrier()

        @pl.loop(0, iters)
        def _(j):
            g = j * NW + wid

            @pl.when(g < nchunks)
            def _():
                pltpu.sync_copy(ei_hbm.at[1, pl.ds(g * CHUNK, CHUNK)], dstv)
                pltpu.sync_copy(onesv, acc.at[dstv], add=True)

        plsc.subcore_barrier()

        @pl.loop(0, blk_iters)
        def _(t):
            b = t * NS + sid

            @pl.when(b < nblk)
            def _():
                r0 = b * ZR
                pltpu.sync_copy(acc.at[pl.ds(r0, ZR)],
                                out_hbm.at[cid, pl.ds(r0, ZR)])

    k = pl.kernel(
        body,
        out_type=jax.ShapeDtypeStruct((NC, n, d), jnp.float32),
        mesh=mesh,
        scratch_types=[
            pltpu.VMEM((CHUNK,), jnp.int32),
            pltpu.VMEM((CHUNK, d), jnp.float32),
            pltpu.VMEM((ZR, d), jnp.float32),
            pltpu.VMEM_SHARED((n, d), jnp.float32),
            pltpu.SemaphoreType.DMA,
        ],
    )
    return k(edge_index)


def _sc_decode_gather(z, eli):
    n, d = z.shape
    el = eli.shape[1]
    per_w = el // NW
    mesh = plsc.VectorSubcoreMesh(core_axis_name="c", subcore_axis_name="s")

    def body(z_hbm, eli_hbm, za_hbm, zb_hbm, idxv, rows, sem):
        cid = lax.axis_index("c")
        sid = lax.axis_index("s")
        wid = sid * NC + cid
        base = wid * per_w
        pltpu.sync_copy(eli_hbm.at[0, pl.ds(base, per_w)], idxv)
        pltpu.async_copy(z_hbm.at[idxv], rows, sem).wait()
        pltpu.sync_copy(rows, za_hbm.at[pl.ds(base, per_w)])
        pltpu.sync_copy(eli_hbm.at[1, pl.ds(base, per_w)], idxv)
        pltpu.async_copy(z_hbm.at[idxv], rows, sem).wait()
        pltpu.sync_copy(rows, zb_hbm.at[pl.ds(base, per_w)])

    k = pl.kernel(
        body,
        out_type=[jax.ShapeDtypeStruct((el, d), jnp.float32)] * 2,
        mesh=mesh,
        scratch_types=[
            pltpu.VMEM((per_w,), jnp.int32),
            pltpu.VMEM((per_w, d), jnp.float32),
            pltpu.SemaphoreType.DMA,
        ],
    )
    return k(z, eli)


_BLK = 1000


def _tc_embed(x, w_hl, b_hl, w_msg0):
    n, d = x.shape

    def body(x_ref, whl_ref, b_ref, wm_ref, h_ref, hm_ref):
        h = jnp.tanh(jnp.dot(x_ref[...], whl_ref[...],
                             preferred_element_type=jnp.float32) + b_ref[...])
        h_ref[...] = h
        hm_ref[...] = jnp.dot(h, wm_ref[...], preferred_element_type=jnp.float32)

    return pl.pallas_call(
        body,
        grid=(n // _BLK,),
        in_specs=[
            pl.BlockSpec((_BLK, d), lambda i: (i, 0)),
            pl.BlockSpec((d, d), lambda i: (0, 0)),
            pl.BlockSpec((1, d), lambda i: (0, 0)),
            pl.BlockSpec((d, d), lambda i: (0, 0)),
        ],
        out_specs=[
            pl.BlockSpec((_BLK, d), lambda i: (i, 0)),
            pl.BlockSpec((_BLK, d), lambda i: (i, 0)),
        ],
        out_shape=[jax.ShapeDtypeStruct((n, d), jnp.float32)] * 2,
    )(x, w_hl, b_hl.reshape(1, d), w_msg0)


def _tc_update(partials, degp, h, w_upd, w_msg):
    n, d = h.shape
    n_out = 1 if w_msg is None else 2

    def body(p_ref, dg_ref, h_ref, wu_ref, *rest):
        if w_msg is None:
            (h1_ref,) = rest
        else:
            wm_ref, h1_ref, hm1_ref = rest
        deg = dg_ref[0, :, 0:1] + dg_ref[1, :, 0:1]
        deg = jnp.maximum(deg, 1.0)
        agg = (p_ref[0] + p_ref[1]) / deg
        h1 = jnp.dot(agg, wu_ref[...], preferred_element_type=jnp.float32) + h_ref[...]
        if w_msg is None:
            h1_ref[...] = h1
        else:
            h1 = jnp.maximum(h1, 0.0)
            h1_ref[...] = h1
            hm1_ref[...] = jnp.dot(h1, wm_ref[...], preferred_element_type=jnp.float32)

    in_specs = [
        pl.BlockSpec((NC, _BLK, d), lambda i: (0, i, 0)),
        pl.BlockSpec((NC, _BLK, d), lambda i: (0, i, 0)),
        pl.BlockSpec((_BLK, d), lambda i: (i, 0)),
        pl.BlockSpec((d, d), lambda i: (0, 0)),
    ]
    args = [partials, degp, h, w_upd]
    if w_msg is not None:
        in_specs.append(pl.BlockSpec((d, d), lambda i: (0, 0)))
        args.append(w_msg)

    return pl.pallas_call(
        body,
        grid=(n // _BLK,),
        in_specs=in_specs,
        out_specs=[pl.BlockSpec((_BLK, d), lambda i: (i, 0))] * n_out,
        out_shape=[jax.ShapeDtypeStruct((n, d), jnp.float32)] * n_out,
    )(*args)


def _tc_dot(za, zb):
    el, d = za.shape

    def body(a_ref, b_ref, s_ref):
        s_ref[...] = jnp.sum(a_ref[...] * b_ref[...], axis=1)

    return pl.pallas_call(
        body,
        out_shape=jax.ShapeDtypeStruct((el,), jnp.float32),
    )(za, zb)


def kernel(x, edge_index, edge_label_index, W_hl, b_hl, W_msg0, W_upd0,
           W_msg1, W_upd1):
    n, d = x.shape
    degp = _sc_degree(edge_index, n, d)
    h, hm0 = _tc_embed(x, W_hl, b_hl, W_msg0)
    p0 = _sc_segment_sum(hm0, edge_index)
    h1, hm1 = _tc_update(p0, degp, h, W_upd0, W_msg1)
    p1 = _sc_segment_sum(hm1, edge_index)
    (z,) = _tc_update(p1, degp, h1, W_upd1, None)
    za, zb = _sc_decode_gather(z, edge_label_index)
    return _tc_dot(za, zb)


# speedup vs baseline: 6.1488x; 1.3259x over previous
import jax
import jax.numpy as jnp
from jax import lax
from jax.experimental import pallas as pl
from jax.experimental.pallas import tpu as pltpu
from jax.experimental.pallas import tpu_sc as plsc

NC = 2
NS = 16
NW = NC * NS
CHUNK = 128
ZR = 8


def _zero_fill(ref, nrows, d):
    zero16 = jnp.zeros((16,), jnp.float32)

    @pl.loop(0, nrows)
    def _(i):
        @pl.loop(0, d // 16)
        def _(c):
            ref[i, pl.ds(c * 16, 16)] = zero16


def _sc_segment_sum(hm, edge_index):
    n, d = hm.shape
    e = edge_index.shape[1]
    nchunks = e // CHUNK
    iters = (nchunks + NW - 1) // NW
    nblk = n // ZR
    blk_iters = (nblk + NS - 1) // NS
    mesh = plsc.VectorSubcoreMesh(core_axis_name="c", subcore_axis_name="s")

    def body(hm_hbm, ei_hbm, out_hbm, srcv0, dstv0, rows0, srcv1, dstv1,
             rows1, zbuf, acc, sem0, sem1):
        cid = lax.axis_index("c")
        sid = lax.axis_index("s")
        wid = sid * NC + cid
        bufs = ((srcv0, dstv0, rows0, sem0), (srcv1, dstv1, rows1, sem1))

        _zero_fill(zbuf, ZR, d)

        @pl.loop(0, blk_iters)
        def _(t):
            b = t * NS + sid

            @pl.when(b < nblk)
            def _():
                pltpu.sync_copy(zbuf, acc.at[pl.ds(b * ZR, ZR)])

        plsc.subcore_barrier()

        def issue(slot, j):
            srcv, dstv, rows, sem = bufs[slot]
            g = j * NW + wid

            @pl.when(g < nchunks)
            def _():
                base = g * CHUNK
                pltpu.sync_copy(ei_hbm.at[0, pl.ds(base, CHUNK)], srcv)
                pltpu.sync_copy(ei_hbm.at[1, pl.ds(base, CHUNK)], dstv)
                pltpu.async_copy(hm_hbm.at[srcv], rows, sem)

        def drain(slot, j):
            srcv, dstv, rows, sem = bufs[slot]
            g = j * NW + wid

            @pl.when(g < nchunks)
            def _():
                pltpu.make_async_copy(hm_hbm.at[srcv], rows, sem).wait()
                pltpu.sync_copy(rows, acc.at[dstv], add=True)

        issue(0, 0)

        @pl.loop(0, (iters + 1) // 2)
        def _(t):
            for b in range(2):
                j = t * 2 + b
                issue((b + 1) % 2, j + 1)
                drain(b, j)

        plsc.subcore_barrier()

        @pl.loop(0, blk_iters)
        def _(t):
            b = t * NS + sid

            @pl.when(b < nblk)
            def _():
                r0 = b * ZR
                pltpu.sync_copy(acc.at[pl.ds(r0, ZR)],
                                out_hbm.at[cid, pl.ds(r0, ZR)])

    k = pl.kernel(
        body,
        out_type=jax.ShapeDtypeStruct((NC, n, d), jnp.float32),
        mesh=mesh,
        scratch_types=[
            pltpu.VMEM((CHUNK,), jnp.int32),
            pltpu.VMEM((CHUNK,), jnp.int32),
            pltpu.VMEM((CHUNK, d), jnp.float32),
            pltpu.VMEM((CHUNK,), jnp.int32),
            pltpu.VMEM((CHUNK,), jnp.int32),
            pltpu.VMEM((CHUNK, d), jnp.float32),
            pltpu.VMEM((ZR, d), jnp.float32),
            pltpu.VMEM_SHARED((n, d), jnp.float32),
            pltpu.SemaphoreType.DMA,
            pltpu.SemaphoreType.DMA,
        ],
    )
    return k(hm, edge_index)


def _sc_degree(edge_index, n, d):
    e = edge_index.shape[1]
    nchunks = e // CHUNK
    iters = (nchunks + NW - 1) // NW
    nblk = n // ZR
    blk_iters = (nblk + NS - 1) // NS
    mesh = plsc.VectorSubcoreMesh(core_axis_name="c", subcore_axis_name="s")

    def body(ei_hbm, out_hbm, dstv, onesv, zbuf, acc, sem):
        cid = lax.axis_index("c")
        sid = lax.axis_index("s")
        wid = sid * NC + cid

        _zero_fill(zbuf, ZR, d)
        one16 = jnp.ones((16,), jnp.float32)

        @pl.loop(0, CHUNK)
        def _(i):
            @pl.loop(0, d // 16)
            def _(c):
                onesv[i, pl.ds(c * 16, 16)] = one16

        @pl.loop(0, blk_iters)
        def _(t):
            b = t * NS + sid

            @pl.when(b < nblk)
            def _():
                pltpu.sync_copy(zbuf, acc.at[pl.ds(b * ZR, ZR)])

        plsc.subcore_barrier()

        @pl.loop(0, iters)
        def _(j):
            g = j * NW + wid

            @pl.when(g < nchunks)
            def _():
                pltpu.sync_copy(ei_hbm.at[1, pl.ds(g * CHUNK, CHUNK)], dstv)
                pltpu.sync_copy(onesv, acc.at[dstv], add=True)

        plsc.subcore_barrier()

        @pl.loop(0, blk_iters)
        def _(t):
            b = t * NS + sid

            @pl.when(b < nblk)
            def _():
                r0 = b * ZR
                pltpu.sync_copy(acc.at[pl.ds(r0, ZR)],
                                out_hbm.at[cid, pl.ds(r0, ZR)])

    k = pl.kernel(
        body,
        out_type=jax.ShapeDtypeStruct((NC, n, d), jnp.float32),
        mesh=mesh,
        scratch_types=[
            pltpu.VMEM((CHUNK,), jnp.int32),
            pltpu.VMEM((CHUNK, d), jnp.float32),
            pltpu.VMEM((ZR, d), jnp.float32),
            pltpu.VMEM_SHARED((n, d), jnp.float32),
            pltpu.SemaphoreType.DMA,
        ],
    )
    return k(edge_index)


def _sc_decode_gather(z, eli):
    n, d = z.shape
    el = eli.shape[1]
    per_w = el // NW
    mesh = plsc.VectorSubcoreMesh(core_axis_name="c", subcore_axis_name="s")

    def body(z_hbm, eli_hbm, za_hbm, zb_hbm, idxv, rows, sem):
        cid = lax.axis_index("c")
        sid = lax.axis_index("s")
        wid = sid * NC + cid
        base = wid * per_w
        pltpu.sync_copy(eli_hbm.at[0, pl.ds(base, per_w)], idxv)
        pltpu.async_copy(z_hbm.at[idxv], rows, sem).wait()
        pltpu.sync_copy(rows, za_hbm.at[pl.ds(base, per_w)])
        pltpu.sync_copy(eli_hbm.at[1, pl.ds(base, per_w)], idxv)
        pltpu.async_copy(z_hbm.at[idxv], rows, sem).wait()
        pltpu.sync_copy(rows, zb_hbm.at[pl.ds(base, per_w)])

    k = pl.kernel(
        body,
        out_type=[jax.ShapeDtypeStruct((el, d), jnp.float32)] * 2,
        mesh=mesh,
        scratch_types=[
            pltpu.VMEM((per_w,), jnp.int32),
            pltpu.VMEM((per_w, d), jnp.float32),
            pltpu.SemaphoreType.DMA,
        ],
    )
    return k(z, eli)


_BLK = 1000


def _tc_embed(x, w_hl, b_hl, w_msg0):
    n, d = x.shape

    def body(x_ref, whl_ref, b_ref, wm_ref, h_ref, hm_ref):
        h = jnp.tanh(jnp.dot(x_ref[...], whl_ref[...],
                             preferred_element_type=jnp.float32) + b_ref[...])
        h_ref[...] = h
        hm_ref[...] = jnp.dot(h, wm_ref[...], preferred_element_type=jnp.float32)

    return pl.pallas_call(
        body,
        grid=(n // _BLK,),
        in_specs=[
            pl.BlockSpec((_BLK, d), lambda i: (i, 0)),
            pl.BlockSpec((d, d), lambda i: (0, 0)),
            pl.BlockSpec((1, d), lambda i: (0, 0)),
            pl.BlockSpec((d, d), lambda i: (0, 0)),
        ],
        out_specs=[
            pl.BlockSpec((_BLK, d), lambda i: (i, 0)),
            pl.BlockSpec((_BLK, d), lambda i: (i, 0)),
        ],
        out_shape=[jax.ShapeDtypeStruct((n, d), jnp.float32)] * 2,
    )(x, w_hl, b_hl.reshape(1, d), w_msg0)


def _tc_update(partials, degp, h, w_upd, w_msg):
    n, d = h.shape
    n_out = 1 if w_msg is None else 2

    def body(p_ref, dg_ref, h_ref, wu_ref, *rest):
        if w_msg is None:
            (h1_ref,) = rest
        else:
            wm_ref, h1_ref, hm1_ref = rest
        deg = dg_ref[0, :, 0:1] + dg_ref[1, :, 0:1]
        deg = jnp.maximum(deg, 1.0)
        agg = (p_ref[0] + p_ref[1]) / deg
        h1 = jnp.dot(agg, wu_ref[...], preferred_element_type=jnp.float32) + h_ref[...]
        if w_msg is None:
            h1_ref[...] = h1
        else:
            h1 = jnp.maximum(h1, 0.0)
            h1_ref[...] = h1
            hm1_ref[...] = jnp.dot(h1, wm_ref[...], preferred_element_type=jnp.float32)

    in_specs = [
        pl.BlockSpec((NC, _BLK, d), lambda i: (0, i, 0)),
        pl.BlockSpec((NC, _BLK, d), lambda i: (0, i, 0)),
        pl.BlockSpec((_BLK, d), lambda i: (i, 0)),
        pl.BlockSpec((d, d), lambda i: (0, 0)),
    ]
    args = [partials, degp, h, w_upd]
    if w_msg is not None:
        in_specs.append(pl.BlockSpec((d, d), lambda i: (0, 0)))
        args.append(w_msg)

    return pl.pallas_call(
        body,
        grid=(n // _BLK,),
        in_specs=in_specs,
        out_specs=[pl.BlockSpec((_BLK, d), lambda i: (i, 0))] * n_out,
        out_shape=[jax.ShapeDtypeStruct((n, d), jnp.float32)] * n_out,
    )(*args)


def _tc_dot(za, zb):
    el, d = za.shape

    def body(a_ref, b_ref, s_ref):
        s_ref[...] = jnp.sum(a_ref[...] * b_ref[...], axis=1)

    return pl.pallas_call(
        body,
        out_shape=jax.ShapeDtypeStruct((el,), jnp.float32),
    )(za, zb)


def kernel(x, edge_index, edge_label_index, W_hl, b_hl, W_msg0, W_upd0,
           W_msg1, W_upd1):
    n, d = x.shape
    degp = _sc_degree(edge_index, n, d)
    h, hm0 = _tc_embed(x, W_hl, b_hl, W_msg0)
    p0 = _sc_segment_sum(hm0, edge_index)
    h1, hm1 = _tc_update(p0, degp, h, W_upd0, W_msg1)
    p1 = _sc_segment_sum(hm1, edge_index)
    (z,) = _tc_update(p1, degp, h1, W_upd1, None)
    za, zb = _sc_decode_gather(z, edge_label_index)
    return _tc_dot(za, zb)
